# baseline (speedup 1.0000x reference)
import functools

import jax
import jax.numpy as jnp
from jax.experimental import pallas as pl
from jax.experimental.pallas import tpu as pltpu

_ROW_BLK = 512


def _feat_mm_kernel(x_ref, w_ref, o_ref):
    o_ref[...] = jnp.dot(x_ref[...], w_ref[...],
                         preferred_element_type=jnp.float32)


def _feat_mm_batched_kernel(x_ref, w_ref, o_ref):
    o_ref[0] = jnp.dot(x_ref[0], w_ref[...],
                       preferred_element_type=jnp.float32)


def _adj_mm_kernel(adj_ref, s_ref, b_ref, o_ref, *, relu, batched_s):
    s = s_ref[0] if batched_s else s_ref[...]
    acc = jnp.dot(adj_ref[0], s, preferred_element_type=jnp.float32)
    acc = acc + b_ref[...]
    if relu:
        acc = jnp.where(acc >= 0, acc, 0.2 * acc)
    o_ref[0] = acc


def _adj_layer(batch, s, bias, *, relu):
    B, N, _ = batch.shape
    D = s.shape[-1]
    n_r = pl.cdiv(N, _ROW_BLK)
    batched_s = s.ndim == 3
    if batched_s:
        s_spec = pl.BlockSpec((1, N, D), lambda b, r: (b, 0, 0))
    else:
        s_spec = pl.BlockSpec((N, D), lambda b, r: (0, 0))
    return pl.pallas_call(
        functools.partial(_adj_mm_kernel, relu=relu, batched_s=batched_s),
        grid=(B, n_r),
        in_specs=[
            pl.BlockSpec((1, _ROW_BLK, N), lambda b, r: (b, r, 0)),
            s_spec,
            pl.BlockSpec((1, D), lambda b, r: (0, 0)),
        ],
        out_specs=pl.BlockSpec((1, _ROW_BLK, D), lambda b, r: (b, r, 0)),
        out_shape=jax.ShapeDtypeStruct((B, N, D), jnp.float32),
        compiler_params=pltpu.CompilerParams(
            dimension_semantics=("parallel", "parallel")),
    )(batch, s, bias)


def kernel(batch, bx, W1, b1, W2, b2):
    B, N, _ = batch.shape
    D = bx.shape[1]
    b1 = b1.reshape(1, D)
    b2 = b2.reshape(1, D)

    s1 = pl.pallas_call(
        _feat_mm_kernel,
        out_shape=jax.ShapeDtypeStruct((N, D), jnp.float32),
    )(bx, W1)

    x1 = _adj_layer(batch, s1, b1, relu=True)

    s2 = pl.pallas_call(
        _feat_mm_batched_kernel,
        grid=(B,),
        in_specs=[
            pl.BlockSpec((1, N, D), lambda b: (b, 0, 0)),
            pl.BlockSpec((D, D), lambda b: (0, 0)),
        ],
        out_specs=pl.BlockSpec((1, N, D), lambda b: (b, 0, 0)),
        out_shape=jax.ShapeDtypeStruct((B, N, D), jnp.float32),
    )(x1, W2)

    return _adj_layer(batch, s2, b2, relu=False)
